# Initial kernel scaffold; baseline (speedup 1.0000x reference)
import jax
import jax.numpy as jnp
from jax.experimental import pallas as pl


def kernel(selection_logit, map2d, offset_gt, tmap):
    raise NotImplementedError("write your pallas kernel here")


# speedup vs baseline: 1.0272x; 1.0272x over previous
import functools

import jax
import jax.numpy as jnp
from jax import lax
from jax.experimental import pallas as pl
from jax.experimental.pallas import tpu as pltpu
from jax.experimental.pallas import tpu_sc as plsc

K = 512
CAP = 640
HK = K // 2
I32_MIN = -(2 ** 31)
PAD_IDX = 1 << 29
MASK31 = 0x7FFFFFFF


def _monotone_key(bits):
    return bits ^ ((bits >> 31) & jnp.int32(MASK31))


def _thresh_body(x_ref, out_ref):
    x = x_ref[...]
    bits = lax.bitcast_convert_type(x, jnp.int32)
    key = _monotone_key(bits)
    b = x.shape[0]

    def cnt_ge(t):
        return jnp.sum((key >= t).astype(jnp.int32), axis=1, keepdims=True)

    zero = jnp.zeros((b, 1), jnp.int32)
    imin = jnp.full((b, 1), I32_MIN, jnp.int32)
    t = jnp.where(cnt_ge(zero) >= K, zero, imin)
    for bit in range(30, -1, -1):
        trial = t | jnp.int32(1 << bit)
        t = jnp.where(cnt_ge(trial) >= K, trial, t)
    tf = lax.bitcast_convert_type(_monotone_key(t), jnp.float32)
    out_ref[...] = jnp.broadcast_to(tf, out_ref.shape)


def _thresh(logit2):
    b = logit2.shape[0]
    out = pl.pallas_call(
        _thresh_body,
        out_shape=jax.ShapeDtypeStruct((b, 128), jnp.float32),
    )(logit2)
    return out[:, 0]


def _compact(logit2, thresh):
    b_total, n = logit2.shape
    mesh = plsc.VectorSubcoreMesh(core_axis_name="c", subcore_axis_name="s")

    @functools.partial(
        pl.kernel,
        out_type=[jax.ShapeDtypeStruct((b_total, CAP), jnp.float32),
                  jax.ShapeDtypeStruct((b_total, CAP), jnp.int32)],
        mesh=mesh,
        compiler_params=pltpu.CompilerParams(needs_layout_passes=False),
        scratch_types=[
            pltpu.VMEM((n,), jnp.float32),
            pltpu.VMEM((CAP,), jnp.float32),
            pltpu.VMEM((CAP,), jnp.int32),
            pltpu.VMEM((32,), jnp.float32),
        ],
    )
    def run(logit_hbm, th_hbm, ckey_hbm, cidx_hbm, vals_v, ckey_v, cidx_v,
            th_v):
        c = lax.axis_index("c")
        s = lax.axis_index("s")

        @pl.when(c == 0)
        def _():
            b = s
            pltpu.sync_copy(logit_hbm.at[b], vals_v)
            pltpu.sync_copy(th_hbm, th_v.at[pl.ds(0, 16)])
            pltpu.sync_copy(th_hbm, th_v.at[pl.ds(16, 16)])
            lane = lax.broadcasted_iota(jnp.int32, (16,), 0)
            tb = th_v[pl.ds(b, 16)][0]

            def pre(i, carry):
                ckey_v[pl.ds(i * 16, 16)] = jnp.full((16,), -jnp.inf,
                                                     jnp.float32)
                cidx_v[pl.ds(i * 16, 16)] = jnp.full((16,), PAD_IDX,
                                                     jnp.int32)
                return carry

            lax.fori_loop(0, CAP // 16, pre, 0)

            def body(i, cnt):
                v = vals_v[pl.ds(i * 16, 16)]
                m = v >= tb
                m = jnp.logical_and(m, cnt <= CAP - 16)
                iv = lane + i * 16
                plsc.store_compressed(ckey_v.at[pl.ds(cnt, 16)], v, mask=m)
                plsc.store_compressed(cidx_v.at[pl.ds(cnt, 16)], iv, mask=m)
                return cnt + plsc.all_reduce_population_count(m)[0]

            lax.fori_loop(0, n // 16, body, jnp.int32(0))
            pltpu.sync_copy(ckey_v, ckey_hbm.at[b])
            pltpu.sync_copy(cidx_v, cidx_hbm.at[b])

    return run(logit2, thresh)


def _rank_body(kr_ref, kc_ref, ir_ref, ic_ref, out_ref):
    krow = kr_ref[0]
    kcol = kc_ref[0]
    irow = ir_ref[0]
    icol = ic_ref[0]
    gt = kcol > krow
    tie = jnp.logical_and(kcol == krow, icol < irow)
    cnt = jnp.sum(jnp.logical_or(gt, tie).astype(jnp.int32), axis=0,
                  keepdims=True)
    out_ref[0] = cnt


def _rank(ckey, cidx):
    b = ckey.shape[0]
    kr = ckey.reshape(b, 1, CAP)
    kc = ckey.reshape(b, CAP, 1)
    ir = cidx.reshape(b, 1, CAP)
    ic = cidx.reshape(b, CAP, 1)
    row_spec = pl.BlockSpec((1, 1, CAP), lambda i: (i, 0, 0))
    col_spec = pl.BlockSpec((1, CAP, 1), lambda i: (i, 0, 0))
    rank3 = pl.pallas_call(
        _rank_body,
        grid=(b,),
        in_specs=[row_spec, col_spec, row_spec, col_spec],
        out_specs=row_spec,
        out_shape=jax.ShapeDtypeStruct((b, 1, CAP), jnp.int32),
    )(kr, kc, ir, ic)
    return rank3.reshape(b, CAP)


def _gather_map(cidx, rank, map_flat):
    b_total = cidx.shape[0]
    n = map_flat.shape[0] // b_total
    d = map_flat.shape[1]
    mesh = plsc.VectorSubcoreMesh(core_axis_name="c", subcore_axis_name="s")

    @functools.partial(
        pl.kernel,
        out_type=[jax.ShapeDtypeStruct((b_total, K, d), jnp.float32),
                  jax.ShapeDtypeStruct((b_total, K, 2), jnp.int32)],
        mesh=mesh,
        compiler_params=pltpu.CompilerParams(needs_layout_passes=False),
        scratch_types=[
            pltpu.VMEM((CAP,), jnp.int32),
            pltpu.VMEM((CAP,), jnp.int32),
            pltpu.VMEM((2, 128), jnp.int32),
            pltpu.VMEM((128, d), jnp.float32),
            pltpu.VMEM((HK, 2), jnp.int32),
            pltpu.SemaphoreType.DMA,
        ],
    )
    def run(cidx_hbm, rank_hbm, map_hbm, prop_hbm, pred_hbm,
            cidx_v, rank_v, g_v, rows_v, pred_v, sem):
        c = lax.axis_index("c")
        s = lax.axis_index("s")
        b = s
        lo = c * HK
        pltpu.sync_copy(cidx_hbm.at[b], cidx_v)
        pltpu.sync_copy(rank_hbm.at[b], rank_v)
        lane = lax.broadcasted_iota(jnp.int32, (16,), 0)

        def sbody(i, carry):
            rk = rank_v[pl.ds(i * 16, 16)]
            ix = cidx_v[pl.ds(i * 16, 16)]
            rrel = rk - lo
            m = jnp.logical_and(rrel >= 0, rrel < HK)
            rsafe = rrel & (HK - 1)
            plsc.store_scatter(g_v, [rsafe >> 7, rsafe & 127], ix + b * n,
                               mask=m)
            return carry

        lax.fori_loop(0, CAP // 16, sbody, 0)

        for j in range(2):
            pltpu.async_copy(map_hbm.at[g_v.at[j]], rows_v, sem).wait()
            pltpu.sync_copy(rows_v, prop_hbm.at[b, pl.ds(lo + j * 128, 128)])

        zeros16 = jnp.zeros((16,), jnp.int32)
        ones16 = jnp.full((16,), 1, jnp.int32)
        for i in range(HK // 16):
            row = g_v.at[i // 8]
            g16 = row[pl.ds((i % 8) * 16, 16)]
            ii = g16 - b * n
            ridx = lane + i * 16
            plsc.store_scatter(pred_v, [ridx, zeros16], ii >> 7)
            plsc.store_scatter(pred_v, [ridx, ones16], (ii & 127) + 1)

        pltpu.sync_copy(pred_v, pred_hbm.at[b, pl.ds(lo, HK)])

    return run(cidx, rank, map_flat)


def _gather_small(cidx, rank, off_flat, tmap_flat):
    b_total = cidx.shape[0]
    n = tmap_flat.shape[0] // b_total
    mesh = plsc.VectorSubcoreMesh(core_axis_name="c", subcore_axis_name="s")

    @functools.partial(
        pl.kernel,
        out_type=[jax.ShapeDtypeStruct((b_total, K, 2), jnp.float32),
                  jax.ShapeDtypeStruct((b_total, K), jnp.float32)],
        mesh=mesh,
        compiler_params=pltpu.CompilerParams(needs_layout_passes=False),
        scratch_types=[
            pltpu.VMEM((CAP,), jnp.int32),
            pltpu.VMEM((CAP,), jnp.int32),
            pltpu.VMEM((HK,), jnp.int32),
            pltpu.VMEM((2 * n,), jnp.float32),
            pltpu.VMEM((n,), jnp.float32),
            pltpu.VMEM((HK, 2), jnp.float32),
            pltpu.VMEM((HK,), jnp.float32),
            pltpu.SemaphoreType.DMA,
        ],
    )
    def run(cidx_hbm, rank_hbm, offf_hbm, tmap_hbm, off_hbm, score_hbm,
            cidx_v, rank_v, ibr_v, offs_stage, tmap_stage, off_v, sc_v, sem):
        c = lax.axis_index("c")
        s = lax.axis_index("s")
        b = s
        lo = c * HK
        stage_cp = [
            pltpu.async_copy(offf_hbm.at[pl.ds(b * 2 * n, 2 * n)],
                             offs_stage, sem),
            pltpu.async_copy(tmap_hbm.at[pl.ds(b * n, n)], tmap_stage, sem),
        ]
        pltpu.sync_copy(cidx_hbm.at[b], cidx_v)
        pltpu.sync_copy(rank_hbm.at[b], rank_v)
        lane = lax.broadcasted_iota(jnp.int32, (16,), 0)

        def sbody(i, carry):
            rk = rank_v[pl.ds(i * 16, 16)]
            ix = cidx_v[pl.ds(i * 16, 16)]
            rrel = rk - lo
            m = jnp.logical_and(rrel >= 0, rrel < HK)
            plsc.store_scatter(ibr_v, [rrel & (HK - 1)], ix, mask=m)
            return carry

        lax.fori_loop(0, CAP // 16, sbody, 0)
        for cp in stage_cp:
            cp.wait()

        zeros16 = jnp.zeros((16,), jnp.int32)
        ones16 = jnp.full((16,), 1, jnp.int32)
        for i in range(HK // 16):
            ii = ibr_v[pl.ds(i * 16, 16)]
            ridx = lane + i * 16
            o0 = plsc.load_gather(offs_stage, [ii * 2])
            o1 = plsc.load_gather(offs_stage, [ii * 2 + 1])
            plsc.store_scatter(off_v, [ridx, zeros16], o0)
            plsc.store_scatter(off_v, [ridx, ones16], o1)
            sc_v[pl.ds(i * 16, 16)] = plsc.load_gather(tmap_stage, [ii])

        out_slice = pl.ds(lo, HK)
        pltpu.sync_copy(off_v, off_hbm.at[b, out_slice])
        pltpu.sync_copy(sc_v, score_hbm.at[b, out_slice])

    return run(cidx, rank, off_flat, tmap_flat)


@jax.jit
def kernel(selection_logit, map2d, offset_gt, tmap):
    b, t, _ = selection_logit.shape
    n = t * t
    d = map2d.shape[-1]
    logit2 = selection_logit.reshape(b, n)
    thresh = _thresh(logit2)
    ckey, cidx = _compact(logit2, thresh)
    rank = _rank(ckey, cidx)
    prop, pred = _gather_map(cidx, rank, map2d.reshape(b * n, d))
    off, score = _gather_small(cidx, rank, offset_gt.reshape(b * n * 2),
                               tmap.reshape(b * n))
    return prop, pred, off, score
